# baseline (speedup 1.0000x reference)
import functools

import jax
import jax.numpy as jnp
from jax import lax
from jax.experimental import pallas as pl
from jax.experimental.pallas import tpu as pltpu
from jax.experimental.pallas import tpu_sc as plsc

_R = 2
_IH = 64
_SPLITS = ((0, 128), (128, 72))
_H = 2
_BB = 8


def _gather_call(rows_pw, idx, table):
    B0, S = idx.shape
    V, D = table.shape
    mesh = plsc.VectorSubcoreMesh(core_axis_name="c", subcore_axis_name="s")
    NC = 2
    n_halves = rows_pw // _IH
    n_chunks = _IH // _R
    n2 = n_chunks // 2
    DP = 2 * D

    @functools.partial(
        pl.kernel,
        out_type=jax.ShapeDtypeStruct((B0, S, DP), jnp.float32),
        mesh=mesh,
        scratch_types=[
            pltpu.VMEM((_IH, S), jnp.int32),
            pltpu.VMEM((2, _R, S, D), jnp.float32),
            pltpu.SemaphoreType.DMA,
            pltpu.SemaphoreType.DMA,
            pltpu.SemaphoreType.DMA,
            pltpu.SemaphoreType.DMA,
        ],
        compiler_params=pltpu.CompilerParams(use_tc_tiling_on_sc=False),
    )
    def emb(idx_hbm, table_hbm, out_hbm, idx_v, rows_v, g0, g1, s0, s1):
        wid = lax.axis_index("s") * NC + lax.axis_index("c")
        b00 = wid * rows_pw
        gsem = (g0, g1)
        ssem = (s0, s1)

        def fire(cc, buf):
            for i in range(_R):
                for (o, w) in _SPLITS:
                    pltpu.async_copy(
                        table_hbm.at[idx_v.at[cc * _R + i, pl.ds(o, w)]],
                        rows_v.at[buf, i, pl.ds(o, w)],
                        gsem[buf],
                    )

        def wait_gathers(buf):
            for i in range(_R):
                for (o, w) in _SPLITS:
                    pltpu.make_async_copy(
                        table_hbm.at[idx_v.at[i, pl.ds(o, w)]],
                        rows_v.at[buf, i, pl.ds(o, w)],
                        gsem[buf],
                    ).wait()

        def fire_store(h, cc, buf):
            b0 = b00 + h * _IH + cc * _R
            pltpu.async_copy(
                rows_v.at[buf],
                out_hbm.at[pl.ds(b0, _R), :, pl.ds(0, D)],
                ssem[buf],
            )

        def wait_store(buf):
            pltpu.make_async_copy(
                rows_v.at[buf],
                out_hbm.at[pl.ds(0, _R), :, pl.ds(0, D)],
                ssem[buf],
            ).wait()

        def half(h, _):
            pltpu.sync_copy(idx_hbm.at[pl.ds(b00 + h * _IH, _IH)], idx_v)
            fire(0, 0)

            def body(i, _):
                @pl.when(jnp.logical_or(i > 0, h > 0))
                def _():
                    wait_store(1)

                fire(2 * i + 1, 1)
                wait_gathers(0)
                fire_store(h, 2 * i, 0)

                @pl.when(i < n2 - 1)
                def _():
                    wait_store(0)
                    fire(2 * i + 2, 0)

                wait_gathers(1)
                fire_store(h, 2 * i + 1, 1)
                return 0

            lax.fori_loop(0, n2, body, 0)
            wait_store(0)
            return 0

        lax.fori_loop(0, n_halves, half, 0)
        wait_store(1)

    return emb(idx, table)


def _finalize_call(prev, part, h, B0, S, D):
    Bh = part.shape[0]
    grid = (Bh // _BB,)
    off = h * (Bh // _BB)

    def body(*refs):
        p_ref, o_ref = refs[-2], refs[-1]
        o_ref[...] = p_ref[:, :, :D]

    in_specs = [
        pl.BlockSpec((_BB, S, 2 * D), lambda i: (i, 0, 0)),
    ]
    args = [part]
    aliases = {}
    if prev is not None:
        in_specs = [pl.BlockSpec(memory_space=pl.ANY)] + in_specs
        args = [prev] + args
        aliases = {0: 0}

        def body(*refs):
            p_ref, o_ref = refs[-2], refs[-1]
            o_ref[...] = p_ref[:, :, :D]

    return pl.pallas_call(
        body,
        grid=grid,
        in_specs=in_specs,
        out_specs=pl.BlockSpec((_BB, S, D), lambda i: (i + off, 0, 0)),
        out_shape=jax.ShapeDtypeStruct((B0, S, D), jnp.float32),
        input_output_aliases=aliases,
    )(*args)


def kernel(input_ids, table):
    B0, S = input_ids.shape
    V, D = table.shape
    NW = 32
    Bh = B0 // _H
    rows_pw = Bh // NW
    assert rows_pw % _IH == 0 and _IH % (2 * _R) == 0

    out = None
    for h in range(_H):
        ids_h = lax.slice_in_dim(input_ids, h * Bh, (h + 1) * Bh, axis=0)
        part = _gather_call(rows_pw, ids_h, table)
        out = _finalize_call(out, part, h, B0, S, D)
    return out


# speedup vs baseline: 2.3016x; 2.3016x over previous
import functools

import jax
import jax.numpy as jnp
from jax import lax
from jax.experimental import pallas as pl
from jax.experimental.pallas import tpu as pltpu
from jax.experimental.pallas import tpu_sc as plsc

_R = 2
_NB = 4
_SPLITS = ((0, 128), (128, 72))


def _emb_call(rows_pw, idx, table):
    B0, S = idx.shape
    V, D = table.shape
    mesh = plsc.VectorSubcoreMesh(core_axis_name="c", subcore_axis_name="s")
    NC = 2
    n_chunks = rows_pw // _R
    n_outer = n_chunks // _NB
    DP = 2 * D

    @functools.partial(
        pl.kernel,
        out_type=jax.ShapeDtypeStruct((B0, S, DP), jnp.float32),
        mesh=mesh,
        scratch_types=[
            pltpu.VMEM((rows_pw, S), jnp.int32),
            pltpu.VMEM((_NB, _R, S, D), jnp.float32),
            [pltpu.SemaphoreType.DMA] * _NB,
            [pltpu.SemaphoreType.DMA] * _NB,
        ],
        compiler_params=pltpu.CompilerParams(use_tc_tiling_on_sc=False),
    )
    def emb(idx_hbm, table_hbm, out_hbm, idx_v, rows_v, gsem, ssem):
        wid = lax.axis_index("s") * NC + lax.axis_index("c")
        b00 = wid * rows_pw

        def fire(cc, buf):
            for i in range(_R):
                for (o, w) in _SPLITS:
                    pltpu.async_copy(
                        table_hbm.at[idx_v.at[cc * _R + i, pl.ds(o, w)]],
                        rows_v.at[buf, i, pl.ds(o, w)],
                        gsem[buf],
                    )

        def wait_gathers(buf):
            for i in range(_R):
                for (o, w) in _SPLITS:
                    pltpu.make_async_copy(
                        table_hbm.at[idx_v.at[i, pl.ds(o, w)]],
                        rows_v.at[buf, i, pl.ds(o, w)],
                        gsem[buf],
                    ).wait()

        def fire_store(cc, buf):
            b0 = b00 + cc * _R
            pltpu.async_copy(
                rows_v.at[buf],
                out_hbm.at[pl.ds(b0, _R), :, pl.ds(0, D)],
                ssem[buf],
            )

        def wait_store(buf):
            pltpu.make_async_copy(
                rows_v.at[buf],
                out_hbm.at[pl.ds(0, _R), :, pl.ds(0, D)],
                ssem[buf],
            ).wait()

        pltpu.sync_copy(idx_hbm.at[pl.ds(b00, rows_pw)], idx_v)
        for b in range(_NB - 1):
            fire(b, b)

        def body(it, _):
            for u in range(_NB):
                cc = _NB * it + u
                nc = cc + (_NB - 1)
                nbuf = (u + _NB - 1) % _NB
                wait_gathers(u)
                fire_store(cc, u)

                @pl.when(nc < n_chunks)
                def _():
                    @pl.when(cc >= 1)
                    def _():
                        wait_store(nbuf)

                    fire(nc, nbuf)

            return 0

        lax.fori_loop(0, n_outer, body, 0)
        for b in range(_NB):
            wait_store(b)

    return emb(idx, table)


def kernel(input_ids, table):
    B0, S = input_ids.shape
    NW = 32
    rows_pw = B0 // NW
    assert rows_pw % (_NB * _R) == 0
    out_p = _emb_call(rows_pw, input_ids, table)
    return out_p[..., : table.shape[1]]
